# baseline (speedup 1.0000x reference)
import functools

import jax
import jax.numpy as jnp
from jax import lax
from jax.experimental import pallas as pl
from jax.experimental.pallas import tpu as pltpu
from jax.experimental.pallas import tpu_sc as plsc

N = 10000
NPAD = 10240
E = 160000
EPAD = 163840
D = 256
H = 128
K = 128
G = 8
ROWS_PER_TILE = NPAD // 16
F32 = jnp.float32

@functools.cache
def _mesh():
    return plsc.VectorSubcoreMesh(
        core_axis_name="c", subcore_axis_name="s", num_cores=2,
        num_subcores=16)


def _tc1_body(x_ref, w_ref, sa_ref, sb_ref):
    xb = x_ref[...]
    w = w_ref[...]
    sa_ref[...] = jnp.dot(xb, w[:, :H], preferred_element_type=F32)
    sb_ref[...] = jnp.dot(xb, w[:, H:], preferred_element_type=F32)


def _tc1(x, w1):
    return pl.pallas_call(
        _tc1_body,
        grid=(10,),
        in_specs=[
            pl.BlockSpec((1000, D), lambda i: (i, 0)),
            pl.BlockSpec((D, D), lambda i: (0, 0)),
        ],
        out_specs=[
            pl.BlockSpec((1000, H), lambda i: (i, 0)),
            pl.BlockSpec((1000, H), lambda i: (i, 0)),
        ],
        out_shape=[
            jax.ShapeDtypeStruct((NPAD, H), F32),
            jax.ShapeDtypeStruct((NPAD, H), F32),
        ],
    )(x, w1)


def _spmm_body(n_chunks, core_chunk_stride, two_tables, g, *refs):
    if two_tables:
        (ta, tb, srcp, dstp, o0, o1,
         acc, rb0, rb1, sA0, sA1, dA0, dA1, s0, s1, sp) = refs
    else:
        (ta, srcp, dstp, o0, o1,
         acc, rb0, rb1, sA0, sA1, dA0, dA1, s0, s1, sp) = refs
        tb = ta
    cid = lax.axis_index("c")
    tid = lax.axis_index("s")

    def _zrow(i, carry):
        def _zcol(j, c2):
            rb0[i, pl.ds(j * 16, 16)] = jnp.zeros((16,), F32)
            return c2
        return lax.fori_loop(0, 8, _zcol, carry)
    lax.fori_loop(0, 128, _zrow, 0)

    base = tid * ROWS_PER_TILE
    for j in range(ROWS_PER_TILE // K):
        pltpu.sync_copy(rb0, acc.at[pl.ds(base + j * K, K)])

    n_stages = n_chunks // g
    crow = cid * core_chunk_stride + tid * n_chunks
    pltpu.sync_copy(srcp.at[pl.ds(crow, g)], sA0)
    pltpu.sync_copy(dstp.at[pl.ds(crow, g)], dA0)
    plsc.subcore_barrier()

    def _gather(idx, rb, sem):
        @pl.when(cid == 0)
        def _():
            pltpu.make_async_copy(
                ta.at[idx.at[pl.ds(0, 64)]], rb.at[pl.ds(0, 64)], sem).start()
            pltpu.make_async_copy(
                ta.at[idx.at[pl.ds(64, 64)]], rb.at[pl.ds(64, 64)],
                sem).start()

        @pl.when(cid == 1)
        def _():
            pltpu.make_async_copy(
                tb.at[idx.at[pl.ds(0, 64)]], rb.at[pl.ds(0, 64)], sem).start()
            pltpu.make_async_copy(
                tb.at[idx.at[pl.ds(64, 64)]], rb.at[pl.ds(64, 64)],
                sem).start()

    def _gwait(rb, sem):
        pltpu.make_async_copy(
            ta.at[sA0.at[0].at[pl.ds(0, 64)]], rb.at[pl.ds(0, 64)],
            sem).wait()
        pltpu.make_async_copy(
            ta.at[sA0.at[0].at[pl.ds(64, 64)]], rb.at[pl.ds(64, 64)],
            sem).wait()

    def _run_stage(sA, dA):
        _gather(sA.at[0], rb0, s0)
        _gather(sA.at[1], rb1, s1)

        def _pair(i, carry):
            k0 = 2 * i
            _gwait(rb0, s0)
            pltpu.sync_copy(rb0, acc.at[dA.at[k0]], add=True)

            @pl.when(k0 + 2 < g)
            def _():
                _gather(sA.at[k0 + 2], rb0, s0)
            _gwait(rb1, s1)
            pltpu.sync_copy(rb1, acc.at[dA.at[k0 + 1]], add=True)

            @pl.when(k0 + 3 < g)
            def _():
                _gather(sA.at[k0 + 3], rb1, s1)
            return carry
        lax.fori_loop(0, g // 2, _pair, 0)

    def _prefetch(stage, sA, dA):
        row = crow + stage * g
        pltpu.make_async_copy(srcp.at[pl.ds(row, g)], sA, sp).start()
        pltpu.make_async_copy(dstp.at[pl.ds(row, g)], dA, sp).start()

    def _pwait(sA, dA):
        pltpu.make_async_copy(srcp.at[pl.ds(crow, g)], sA, sp).wait()
        pltpu.make_async_copy(dstp.at[pl.ds(crow, g)], dA, sp).wait()

    def _outer(t, carry):
        s_even = 2 * t
        _prefetch(s_even + 1, sA1, dA1)
        _run_stage(sA0, dA0)
        _pwait(sA1, dA1)

        @pl.when(s_even + 2 < n_stages)
        def _():
            _prefetch(s_even + 2, sA0, dA0)
        _run_stage(sA1, dA1)

        @pl.when(s_even + 2 < n_stages)
        def _():
            _pwait(sA0, dA0)
        return carry
    lax.fori_loop(0, n_stages // 2, _outer, 0)
    if n_stages % 2:
        _run_stage(sA0, dA0)
    plsc.subcore_barrier()

    @pl.when(cid == 0)
    def _():
        pltpu.sync_copy(acc.at[pl.ds(base, ROWS_PER_TILE)],
                        o0.at[pl.ds(base, ROWS_PER_TILE)])

    @pl.when(cid == 1)
    def _():
        pltpu.sync_copy(acc.at[pl.ds(base, ROWS_PER_TILE)],
                        o1.at[pl.ds(base, ROWS_PER_TILE)])


def _make_spmm(n_chunks, core_chunk_stride, two_tables, g=G):
    body = functools.partial(_spmm_body, n_chunks, core_chunk_stride,
                             two_tables, g)
    return pl.kernel(
        body,
        out_type=[
            jax.ShapeDtypeStruct((NPAD, H), F32),
            jax.ShapeDtypeStruct((NPAD, H), F32),
        ],
        mesh=_mesh(),
        scratch_types=[
            pltpu.VMEM_SHARED((NPAD, H), F32),
            pltpu.VMEM((K, H), F32),
            pltpu.VMEM((K, H), F32),
            pltpu.VMEM((g, K), jnp.int32),
            pltpu.VMEM((g, K), jnp.int32),
            pltpu.VMEM((g, K), jnp.int32),
            pltpu.VMEM((g, K), jnp.int32),
            pltpu.SemaphoreType.DMA,
            pltpu.SemaphoreType.DMA,
            pltpu.SemaphoreType.DMA,
        ],
    )


def _tc2_body(h0_ref, h1_ref, b1_ref, w2_ref, s2_ref):
    a0 = jnp.maximum(h0_ref[...] + b1_ref[0:1, :], 0.0)
    a1 = jnp.maximum(h1_ref[...] + b1_ref[1:2, :], 0.0)
    s2_ref[...] = (jnp.dot(a0, w2_ref[:H, :], preferred_element_type=F32)
                   + jnp.dot(a1, w2_ref[H:, :], preferred_element_type=F32))


def _tc2(h0, h1, b1r, w2p):
    return pl.pallas_call(
        _tc2_body,
        grid=(10,),
        in_specs=[
            pl.BlockSpec((1000, H), lambda i: (i, 0)),
            pl.BlockSpec((1000, H), lambda i: (i, 0)),
            pl.BlockSpec((2, H), lambda i: (0, 0)),
            pl.BlockSpec((D, H), lambda i: (0, 0)),
        ],
        out_specs=pl.BlockSpec((1000, H), lambda i: (i, 0)),
        out_shape=jax.ShapeDtypeStruct((NPAD, H), F32),
    )(h0, h1, b1r, w2p)


def _tc3_body(p0_ref, p1_ref, b2_ref, out_ref):
    v = p0_ref[...] + p1_ref[...] + b2_ref[...]
    mask = lax.broadcasted_iota(jnp.int32, v.shape, 1) < 40
    vm = jnp.where(mask, v, jnp.float32(-1e30))
    mx = jnp.max(vm, axis=1, keepdims=True)
    ex = jnp.where(mask, jnp.exp(v - mx), 0.0)
    lse = jnp.log(jnp.sum(ex, axis=1, keepdims=True))
    res = v - mx - lse
    out_ref[...] = res[:, :40]


def _tc3(p0, p1, b2p):
    return pl.pallas_call(
        _tc3_body,
        grid=(10,),
        in_specs=[
            pl.BlockSpec((1000, H), lambda i: (i, 0)),
            pl.BlockSpec((1000, H), lambda i: (i, 0)),
            pl.BlockSpec((1, H), lambda i: (0, 0)),
        ],
        out_specs=pl.BlockSpec((1000, 40), lambda i: (i, 0)),
        out_shape=jax.ShapeDtypeStruct((N, 40), F32),
    )(p0, p1, b2p)


def kernel(x, edge_index, W1, b1, W2, b2):
    src = edge_index[0]
    dst = edge_index[1]
    pad = EPAD - E
    src_p = jnp.concatenate(
        [src, (jnp.arange(pad, dtype=jnp.int32) * 13) % N])
    dst_p = jnp.concatenate(
        [dst, N + (jnp.arange(pad, dtype=jnp.int32) % (NPAD - N))])
    src_p = src_p.reshape(EPAD // K, K)
    dst_p = dst_p.reshape(EPAD // K, K)
    w2p = jnp.zeros((D, H), F32).at[:, :40].set(W2)
    b1r = b1.reshape(2, H)
    b2p = jnp.zeros((1, H), F32).at[0, :40].set(b2)

    sa, sb = _tc1(x, W1)
    h0, h1 = _make_spmm(EPAD // 16 // K, 0, True, g=16)(sa, sb, src_p, dst_p)
    s2 = _tc2(h0, h1, b1r, w2p)
    p0, p1 = _make_spmm(EPAD // 2 // 16 // K, EPAD // 2 // K,
                        False)(s2, src_p, dst_p)
    return _tc3(p0, p1, b2p)


# speedup vs baseline: 1.0221x; 1.0221x over previous
import functools

import jax
import jax.numpy as jnp
from jax import lax
from jax.experimental import pallas as pl
from jax.experimental.pallas import tpu as pltpu
from jax.experimental.pallas import tpu_sc as plsc

N = 10000
NPAD = 10240
E = 160000
EPAD = 163840
D = 256
H = 128
K = 128
G = 8
ROWS_PER_TILE = NPAD // 16
F32 = jnp.float32

@functools.cache
def _mesh():
    return plsc.VectorSubcoreMesh(
        core_axis_name="c", subcore_axis_name="s", num_cores=2,
        num_subcores=16)


def _tc1_body(x_ref, w_ref, sa_ref, sb_ref):
    xb = x_ref[...]
    w = w_ref[...]
    sa_ref[...] = jnp.dot(xb, w[:, :H], preferred_element_type=F32)
    sb_ref[...] = jnp.dot(xb, w[:, H:], preferred_element_type=F32)


def _tc1(x, w1):
    return pl.pallas_call(
        _tc1_body,
        grid=(10,),
        in_specs=[
            pl.BlockSpec((1000, D), lambda i: (i, 0)),
            pl.BlockSpec((D, D), lambda i: (0, 0)),
        ],
        out_specs=[
            pl.BlockSpec((1000, H), lambda i: (i, 0)),
            pl.BlockSpec((1000, H), lambda i: (i, 0)),
        ],
        out_shape=[
            jax.ShapeDtypeStruct((NPAD, H), F32),
            jax.ShapeDtypeStruct((NPAD, H), F32),
        ],
    )(x, w1)


def _spmm_body(n_chunks, core_chunk_stride, two_tables, g, *refs):
    if two_tables:
        (ta, tb, srcp, dstp, o0, o1,
         acc, rb0, rb1, sA0, sA1, dA0, dA1, s0, s1, sp) = refs
    else:
        (ta, srcp, dstp, o0, o1,
         acc, rb0, rb1, sA0, sA1, dA0, dA1, s0, s1, sp) = refs
        tb = ta
    cid = lax.axis_index("c")
    tid = lax.axis_index("s")

    def _zrow(i, carry):
        def _zcol(j, c2):
            rb0[i, pl.ds(j * 16, 16)] = jnp.zeros((16,), F32)
            return c2
        return lax.fori_loop(0, 8, _zcol, carry)
    lax.fori_loop(0, 128, _zrow, 0)

    base = tid * ROWS_PER_TILE
    for j in range(ROWS_PER_TILE // K):
        pltpu.sync_copy(rb0, acc.at[pl.ds(base + j * K, K)])

    n_stages = n_chunks // g
    crow = cid * core_chunk_stride + tid * n_chunks
    pltpu.sync_copy(srcp.at[pl.ds(crow, g)], sA0)
    pltpu.sync_copy(dstp.at[pl.ds(crow, g)], dA0)
    plsc.subcore_barrier()

    def _gather(idx, rb, sem):
        @pl.when(cid == 0)
        def _():
            pltpu.make_async_copy(ta.at[idx], rb, sem).start()

        @pl.when(cid == 1)
        def _():
            pltpu.make_async_copy(tb.at[idx], rb, sem).start()

    def _gwait(rb, sem):
        pltpu.make_async_copy(ta.at[sA0.at[0]], rb, sem).wait()

    def _run_stage(sA, dA):
        _gather(sA.at[0], rb0, s0)
        _gather(sA.at[1], rb1, s1)

        def _pair(i, carry):
            k0 = 2 * i
            _gwait(rb0, s0)
            pltpu.sync_copy(rb0, acc.at[dA.at[k0]], add=True)

            @pl.when(k0 + 2 < g)
            def _():
                _gather(sA.at[k0 + 2], rb0, s0)
            _gwait(rb1, s1)
            pltpu.sync_copy(rb1, acc.at[dA.at[k0 + 1]], add=True)

            @pl.when(k0 + 3 < g)
            def _():
                _gather(sA.at[k0 + 3], rb1, s1)
            return carry
        lax.fori_loop(0, g // 2, _pair, 0)

    def _prefetch(stage, sA, dA):
        row = crow + stage * g
        pltpu.make_async_copy(srcp.at[pl.ds(row, g)], sA, sp).start()
        pltpu.make_async_copy(dstp.at[pl.ds(row, g)], dA, sp).start()

    def _pwait(sA, dA):
        pltpu.make_async_copy(srcp.at[pl.ds(crow, g)], sA, sp).wait()
        pltpu.make_async_copy(dstp.at[pl.ds(crow, g)], dA, sp).wait()

    def _outer(t, carry):
        s_even = 2 * t
        _prefetch(s_even + 1, sA1, dA1)
        _run_stage(sA0, dA0)
        _pwait(sA1, dA1)

        @pl.when(s_even + 2 < n_stages)
        def _():
            _prefetch(s_even + 2, sA0, dA0)
        _run_stage(sA1, dA1)

        @pl.when(s_even + 2 < n_stages)
        def _():
            _pwait(sA0, dA0)
        return carry
    lax.fori_loop(0, n_stages // 2, _outer, 0)
    if n_stages % 2:
        _run_stage(sA0, dA0)
    plsc.subcore_barrier()

    @pl.when(cid == 0)
    def _():
        pltpu.sync_copy(acc.at[pl.ds(base, ROWS_PER_TILE)],
                        o0.at[pl.ds(base, ROWS_PER_TILE)])

    @pl.when(cid == 1)
    def _():
        pltpu.sync_copy(acc.at[pl.ds(base, ROWS_PER_TILE)],
                        o1.at[pl.ds(base, ROWS_PER_TILE)])


def _make_spmm(n_chunks, core_chunk_stride, two_tables, g=G):
    body = functools.partial(_spmm_body, n_chunks, core_chunk_stride,
                             two_tables, g)
    g2 = g if n_chunks // g > 1 else 1
    return pl.kernel(
        body,
        out_type=[
            jax.ShapeDtypeStruct((NPAD, H), F32),
            jax.ShapeDtypeStruct((NPAD, H), F32),
        ],
        mesh=_mesh(),
        scratch_types=[
            pltpu.VMEM_SHARED((NPAD, H), F32),
            pltpu.VMEM((K, H), F32),
            pltpu.VMEM((K, H), F32),
            pltpu.VMEM((g, K), jnp.int32),
            pltpu.VMEM((g2, K), jnp.int32),
            pltpu.VMEM((g, K), jnp.int32),
            pltpu.VMEM((g2, K), jnp.int32),
            pltpu.SemaphoreType.DMA,
            pltpu.SemaphoreType.DMA,
            pltpu.SemaphoreType.DMA,
        ],
    )


def _tc2_body(h0_ref, h1_ref, b1_ref, w2_ref, s2_ref):
    a0 = jnp.maximum(h0_ref[...] + b1_ref[0:1, :], 0.0)
    a1 = jnp.maximum(h1_ref[...] + b1_ref[1:2, :], 0.0)
    s2_ref[...] = (jnp.dot(a0, w2_ref[:H, :], preferred_element_type=F32)
                   + jnp.dot(a1, w2_ref[H:, :], preferred_element_type=F32))


def _tc2(h0, h1, b1r, w2p):
    return pl.pallas_call(
        _tc2_body,
        grid=(10,),
        in_specs=[
            pl.BlockSpec((1000, H), lambda i: (i, 0)),
            pl.BlockSpec((1000, H), lambda i: (i, 0)),
            pl.BlockSpec((2, H), lambda i: (0, 0)),
            pl.BlockSpec((D, H), lambda i: (0, 0)),
        ],
        out_specs=pl.BlockSpec((1000, H), lambda i: (i, 0)),
        out_shape=jax.ShapeDtypeStruct((NPAD, H), F32),
    )(h0, h1, b1r, w2p)


def _tc3_body(p0_ref, p1_ref, b2_ref, out_ref):
    v = p0_ref[...] + p1_ref[...] + b2_ref[...]
    mask = lax.broadcasted_iota(jnp.int32, v.shape, 1) < 40
    vm = jnp.where(mask, v, jnp.float32(-1e30))
    mx = jnp.max(vm, axis=1, keepdims=True)
    ex = jnp.where(mask, jnp.exp(v - mx), 0.0)
    lse = jnp.log(jnp.sum(ex, axis=1, keepdims=True))
    res = v - mx - lse
    out_ref[...] = res[:, :40]


def _tc3(p0, p1, b2p):
    return pl.pallas_call(
        _tc3_body,
        grid=(10,),
        in_specs=[
            pl.BlockSpec((1000, H), lambda i: (i, 0)),
            pl.BlockSpec((1000, H), lambda i: (i, 0)),
            pl.BlockSpec((1, H), lambda i: (0, 0)),
        ],
        out_specs=pl.BlockSpec((1000, 40), lambda i: (i, 0)),
        out_shape=jax.ShapeDtypeStruct((N, 40), F32),
    )(p0, p1, b2p)


def kernel(x, edge_index, W1, b1, W2, b2):
    src = edge_index[0]
    dst = edge_index[1]
    pad = EPAD - E
    src_p = jnp.concatenate(
        [src, (jnp.arange(pad, dtype=jnp.int32) * 13) % N])
    dst_p = jnp.concatenate(
        [dst, N + (jnp.arange(pad, dtype=jnp.int32) % (NPAD - N))])
    src_p = src_p.reshape(EPAD // K, K)
    dst_p = dst_p.reshape(EPAD // K, K)
    w2p = jnp.zeros((D, H), F32).at[:, :40].set(W2)
    b1r = b1.reshape(2, H)
    b2p = jnp.zeros((1, H), F32).at[0, :40].set(b2)

    sa, sb = _tc1(x, W1)
    h0, h1 = _make_spmm(EPAD // 16 // K, 0, True, g=16)(sa, sb, src_p, dst_p)
    s2 = _tc2(h0, h1, b1r, w2p)
    p0, p1 = _make_spmm(EPAD // 2 // 16 // K, EPAD // 2 // K,
                        False, g=40)(s2, src_p, dst_p)
    return _tc3(p0, p1, b2p)
